# Initial kernel scaffold; baseline (speedup 1.0000x reference)
import jax
import jax.numpy as jnp
from jax.experimental import pallas as pl


def kernel(X_op, X_m, E_seq, E_op2m, Wop0, bop0, Wm0, bm0, Wop1, bop1, Wm1, bm1, Wop2, bop2, Wm2, bm2, HW1, Hb1, HW2, Hb2):
    raise NotImplementedError("write your pallas kernel here")


# speedup vs baseline: 3.2919x; 3.2919x over previous
import functools

import jax
import jax.numpy as jnp
from jax import lax
from jax.experimental import pallas as pl
from jax.experimental.pallas import tpu as pltpu
from jax.experimental.pallas import tpu_sc as plsc

N = 10000
F_IN = 256
HID = 512
CW = 128
NCH = HID // CW
E = 160000
EB = 128
NB_E = E // EB
E_PAD = 163840
NBT = E_PAD // (16 * EB)
NS = 16
RPT = N // NS
DEG_N = 10240
BM = 1000

f32 = jnp.float32
i32 = jnp.int32


def _sc_mesh():
    return plsc.VectorSubcoreMesh(core_axis_name="c", subcore_axis_name="s",
                                  num_cores=2, num_subcores=NS)


def _deg_body(i1, i2, i3, ones_h, zeros_h, o1, o2, o3,
              idxbuf, onesbuf, stage, dacc):
    c = lax.axis_index("c")
    s = lax.axis_index("s")
    pltpu.sync_copy(ones_h, onesbuf)
    nb = jnp.where(s < 2, 79, 78)
    base = pl.multiple_of(s * 632, 8)
    spans_main = ((0, 128), (128, 128), (256, 128), (384, 128), (512, 120))
    spans_last = ((0, 128), (128, 128), (256, 128), (384, 128), (512, 8))
    zspan_last = ((0, 128), (128, 128), (256, 128), (384, 128), (512, 16))

    def _for_spans(spl, fn):
        @pl.when(s < 15)
        def _():
            for r0, nr in spans_main:
                fn(r0, nr)

        @pl.when(s == 15)
        def _():
            for r0, nr in spl:
                fn(r0, nr)

    for lst, o, core in ((i1, o1, 0), (i2, o2, 1), (i3, o3, 0)):
        @pl.when(c == core)
        def _():
            pltpu.sync_copy(zeros_h, stage)

            def zero_fn(r0, nr):
                pltpu.sync_copy(stage.at[pl.ds(0, nr)],
                                acc_slice(r0, nr))

            def acc_slice(r0, nr):
                return dacc.at[pl.ds(base + r0, nr)]

            _for_spans(zspan_last, zero_fn)
            plsc.subcore_barrier()

            def bbody(i, carry):
                b = s + NS * i
                boff = pl.multiple_of(b * EB, EB)
                pltpu.sync_copy(lst.at[pl.ds(boff, EB)], idxbuf)
                pltpu.sync_copy(onesbuf, dacc.at[idxbuf], add=True)
                return carry

            lax.fori_loop(0, nb, bbody, 0)
            plsc.subcore_barrier()

            def write_fn(r0, nr):
                pltpu.sync_copy(dacc.at[pl.ds(base + r0, nr)],
                                stage.at[pl.ds(0, nr)])
                pltpu.sync_copy(stage.at[pl.ds(0, nr)],
                                o.at[pl.ds(base + r0, nr)])

            _for_spans(spans_last, write_fn)
            plsc.subcore_barrier()


_deg_kernel = pl.kernel(
    _deg_body,
    out_type=(jax.ShapeDtypeStruct((DEG_N, CW), f32),) * 3,
    mesh=_sc_mesh(),
    scratch_types=[
        pltpu.VMEM((EB,), i32),
        pltpu.VMEM((EB, CW), f32),
        pltpu.VMEM((EB, CW), f32),
        pltpu.VMEM_SHARED((N + 8, CW), f32),
    ],
)


def _agg_body(table, ed2, z, out, pb0, pb1, rows, acc, sem, l0, l1):
    c = lax.axis_index("c")
    s = lax.axis_index("s")
    pbuf = (pb0, pb1)
    lsem = (l0, l1)
    nb = jnp.where(s < 2, 79, 78)
    base = pl.multiple_of(s * 632, 8)

    def _rows_split(fn):
        @pl.when(s < 15)
        def _():
            fn(632)

        @pl.when(s == 15)
        def _():
            fn(520)

    for j in range(2):
        ch = 2 * c + j

        def zero_fn(nr):
            pltpu.sync_copy(z.at[pl.ds(0, nr)], acc.at[pl.ds(base, nr)])

        _rows_split(zero_fn)
        plsc.subcore_barrier()
        off = ch * N

        pltpu.async_copy(ed2.at[s], pb0, l0)

        def bbody(t, carry):
            for b in range(2):
                i = 2 * t + b

                @pl.when(i < nb)
                def _():
                    pltpu.make_async_copy(ed2.at[s], pbuf[b], lsem[b]).wait()

                    @pl.when(i + 1 < nb)
                    def _():
                        pltpu.async_copy(ed2.at[s + NS * (i + 1)],
                                         pbuf[1 - b], lsem[1 - b])

                    for v in range(8):
                        pbuf[b][0, pl.ds(v * 16, 16)] = (
                            pbuf[b][0, pl.ds(v * 16, 16)] + off)
                    pltpu.async_copy(table.at[pbuf[b].at[0]], rows,
                                     sem).wait()
                    pltpu.sync_copy(rows, acc.at[pbuf[b].at[1]], add=True)
            return carry

        lax.fori_loop(0, 40, bbody, 0)
        plsc.subcore_barrier()
        obase = pl.multiple_of(ch * N, 8) + base

        def write_fn(nr):
            pltpu.sync_copy(acc.at[pl.ds(base, nr)],
                            out.at[pl.ds(obase, nr)])

        _rows_split(write_fn)
        plsc.subcore_barrier()


_agg_kernel = pl.kernel(
    _agg_body,
    out_type=jax.ShapeDtypeStruct((NCH * N, CW), f32),
    mesh=_sc_mesh(),
    scratch_types=[
        pltpu.VMEM((2, EB), i32),
        pltpu.VMEM((2, EB), i32),
        pltpu.VMEM((EB, CW), f32),
        pltpu.VMEM_SHARED((N, CW), f32),
        pltpu.SemaphoreType.DMA,
        pltpu.SemaphoreType.DMA,
        pltpu.SemaphoreType.DMA,
    ],
)


def _mm_body(a, w, b, o, *, nk):
    av = jnp.concatenate([a[k] for k in range(nk)], axis=1)
    acc = jnp.dot(av, w[...], preferred_element_type=f32)
    o[0] = acc + b[0]


def _mm(a3, w, b2):
    nk = a3.shape[0]
    return pl.pallas_call(
        functools.partial(_mm_body, nk=nk),
        grid=(N // BM, NCH),
        in_specs=[
            pl.BlockSpec((nk, BM, CW), lambda i, c: (0, i, 0)),
            pl.BlockSpec((nk * CW, CW), lambda i, c: (0, c)),
            pl.BlockSpec((1, 1, CW), lambda i, c: (c, 0, 0)),
        ],
        out_specs=pl.BlockSpec((1, BM, CW), lambda i, c: (c, i, 0)),
        out_shape=jax.ShapeDtypeStruct((NCH, N, CW), f32),
    )(a3, w, b2)


def _ew_op_body(hp, a1, d1, a3, d3, o):
    o[0] = jnp.maximum(hp[0] + a1[0] / jnp.maximum(d1[...], 1.0)
                       + a3[0] / jnp.maximum(d3[...], 1.0), 0.0)


def _ew_op(hp, a1, d1, a3, d3):
    return pl.pallas_call(
        _ew_op_body,
        grid=(N // BM, NCH),
        in_specs=[
            pl.BlockSpec((1, BM, CW), lambda i, c: (c, i, 0)),
            pl.BlockSpec((1, BM, CW), lambda i, c: (c, i, 0)),
            pl.BlockSpec((BM, CW), lambda i, c: (i, 0)),
            pl.BlockSpec((1, BM, CW), lambda i, c: (c, i, 0)),
            pl.BlockSpec((BM, CW), lambda i, c: (i, 0)),
        ],
        out_specs=pl.BlockSpec((1, BM, CW), lambda i, c: (c, i, 0)),
        out_shape=jax.ShapeDtypeStruct((NCH, N, CW), f32),
    )(hp, a1, d1, a3, d3)


def _ew_m_body(hp, a2, d2, o):
    o[0] = jnp.maximum(hp[0] + a2[0] / jnp.maximum(d2[...], 1.0), 0.0)


def _ew_m(hp, a2, d2):
    return pl.pallas_call(
        _ew_m_body,
        grid=(N // BM, NCH),
        in_specs=[
            pl.BlockSpec((1, BM, CW), lambda i, c: (c, i, 0)),
            pl.BlockSpec((1, BM, CW), lambda i, c: (c, i, 0)),
            pl.BlockSpec((BM, CW), lambda i, c: (i, 0)),
        ],
        out_specs=pl.BlockSpec((1, BM, CW), lambda i, c: (c, i, 0)),
        out_shape=jax.ShapeDtypeStruct((NCH, N, CW), f32),
    )(hp, a2, d2)


def _ew_sum_body(*args, nin):
    i = pl.program_id(1)
    if nin == 5:
        hp, a1, d1, a3, d3, osum = args
        h = jnp.maximum(hp[0] + a1[0] / jnp.maximum(d1[...], 1.0)
                        + a3[0] / jnp.maximum(d3[...], 1.0), 0.0)
    else:
        hp, a2, d2, osum = args
        h = jnp.maximum(hp[0] + a2[0] / jnp.maximum(d2[...], 1.0), 0.0)
    s2 = jnp.sum(h, axis=0, keepdims=True)

    @pl.when(i == 0)
    def _():
        osum[0] = s2

    @pl.when(i > 0)
    def _():
        osum[0] += s2


def _ew_sum(hp, aggs_and_degs):
    nin = 1 + len(aggs_and_degs)
    big = pl.BlockSpec((1, BM, CW), lambda c, i: (c, i, 0))
    deg = pl.BlockSpec((BM, CW), lambda c, i: (i, 0))
    specs = [big] + [big if k % 2 == 0 else deg
                     for k in range(len(aggs_and_degs))]
    return pl.pallas_call(
        functools.partial(_ew_sum_body, nin=nin),
        grid=(NCH, N // BM),
        in_specs=specs,
        out_specs=pl.BlockSpec((1, 1, CW), lambda c, i: (c, 0, 0)),
        out_shape=jax.ShapeDtypeStruct((NCH, 1, CW), f32),
    )(hp, *aggs_and_degs)


def _head_body(so, sm, w1, b1, w2, b2, o):
    acc = jnp.zeros((1, HID), f32)
    scale = 1.0 / N
    for k in range(NCH):
        acc += jnp.dot(so[pl.ds(k, 1), :] * scale, w1[pl.ds(k * CW, CW), :],
                       preferred_element_type=f32)
        acc += jnp.dot(sm[pl.ds(k, 1), :] * scale,
                       w1[pl.ds(HID + k * CW, CW), :],
                       preferred_element_type=f32)
    h = jnp.maximum(acc + b1[...], 0.0)
    o[...] = jnp.dot(h, w2[...], preferred_element_type=f32) + b2[...]


def _head(so, sm, w1, b1, w2p, b2p):
    return pl.pallas_call(
        _head_body,
        out_shape=jax.ShapeDtypeStruct((1, CW), f32),
    )(so, sm, w1, b1, w2p, b2p)


def kernel(X_op, X_m, E_seq, E_op2m, Wop0, bop0, Wm0, bm0, Wop1, bop1,
           Wm1, bm1, Wop2, bop2, Wm2, bm2, HW1, Hb1, HW2, Hb2):
    src_seq = E_seq[0]
    dst_seq = E_seq[1]
    src_op = E_op2m[0]
    dst_m = E_op2m[1]

    def _pair(a, b):
        return jnp.stack([a.reshape(NB_E, EB), b.reshape(NB_E, EB)], axis=1)

    ed_1 = _pair(src_seq, dst_seq)
    ed_2 = _pair(src_op, dst_m)
    ed_3 = _pair(dst_m, src_op)

    z128 = jnp.zeros((632, CW), f32)
    ones_h = jnp.ones((EB, CW), f32)
    zeros_h = jnp.zeros((EB, CW), f32)

    g1, g2, g3 = _deg_kernel(dst_seq, dst_m, src_op, ones_h, zeros_h)
    d1b = jnp.broadcast_to(g1[:N, :1], (N, CW))
    d2b = jnp.broadcast_to(g2[:N, :1], (N, CW))
    d3b = jnp.broadcast_to(g3[:N, :1], (N, CW))

    Hop = X_op.reshape(N, 2, CW).transpose(1, 0, 2)
    Hm = X_m.reshape(N, 2, CW).transpose(1, 0, 2)
    params = [(Wop0, bop0, Wm0, bm0), (Wop1, bop1, Wm1, bm1),
              (Wop2, bop2, Wm2, bm2)]

    for li, (Wo, bo, Wm_, bm_) in enumerate(params):
        Hp_op = _mm(Hop, Wo, bo.reshape(NCH, 1, CW))
        Hp_m = _mm(Hm, Wm_, bm_.reshape(NCH, 1, CW))
        tbl_op = Hp_op.reshape(NCH * N, CW)
        tbl_m = Hp_m.reshape(NCH * N, CW)
        a1 = _agg_kernel(tbl_op, ed_1, z128).reshape(NCH, N, CW)
        a2 = _agg_kernel(tbl_op, ed_2, z128).reshape(NCH, N, CW)
        a3 = _agg_kernel(tbl_m, ed_3, z128).reshape(NCH, N, CW)
        if li < 2:
            Hop = _ew_op(Hp_op, a1, d1b, a3, d3b)
            Hm = _ew_m(Hp_m, a2, d2b)
        else:
            s_op = _ew_sum(Hp_op, (a1, d1b, a3, d3b)).reshape(NCH, CW)
            s_m = _ew_sum(Hp_m, (a2, d2b)).reshape(NCH, CW)

    w2p = jnp.pad(HW2, ((0, 0), (0, CW - 1)))
    b2p = jnp.pad(Hb2, (0, CW - 1)).reshape(1, CW)
    y = _head(s_op, s_m, HW1, Hb1.reshape(1, HID), w2p, b2p)
    return y[0, :1]


# speedup vs baseline: 4.2508x; 1.2913x over previous
import functools

import jax
import jax.numpy as jnp
from jax import lax
from jax.experimental import pallas as pl
from jax.experimental.pallas import tpu as pltpu
from jax.experimental.pallas import tpu_sc as plsc

N = 10000
F_IN = 256
HID = 512
CW = 128
NCH = HID // CW
E = 160000
EB = 128
NB_E = E // EB
E_PAD = 163840
NBT = E_PAD // (16 * EB)
NS = 16
RPT = N // NS
DEG_N = 10240
BM = 1000

f32 = jnp.float32
i32 = jnp.int32


def _sc_mesh():
    return plsc.VectorSubcoreMesh(core_axis_name="c", subcore_axis_name="s",
                                  num_cores=2, num_subcores=NS)


def _deg_body(i1, i2, i3, ones_h, zeros_h, o1, o2, o3,
              idxbuf, onesbuf, stage, dacc):
    c = lax.axis_index("c")
    s = lax.axis_index("s")
    pltpu.sync_copy(ones_h, onesbuf)
    nb = jnp.where(s < 2, 79, 78)
    base = pl.multiple_of(s * 632, 8)
    spans_main = ((0, 128), (128, 128), (256, 128), (384, 128), (512, 120))
    spans_last = ((0, 128), (128, 128), (256, 128), (384, 128), (512, 8))
    zspan_last = ((0, 128), (128, 128), (256, 128), (384, 128), (512, 16))

    def _for_spans(spl, fn):
        @pl.when(s < 15)
        def _():
            for r0, nr in spans_main:
                fn(r0, nr)

        @pl.when(s == 15)
        def _():
            for r0, nr in spl:
                fn(r0, nr)

    for lst, o, core in ((i1, o1, 0), (i2, o2, 1), (i3, o3, 0)):
        @pl.when(c == core)
        def _():
            pltpu.sync_copy(zeros_h, stage)

            def zero_fn(r0, nr):
                pltpu.sync_copy(stage.at[pl.ds(0, nr)],
                                acc_slice(r0, nr))

            def acc_slice(r0, nr):
                return dacc.at[pl.ds(base + r0, nr)]

            _for_spans(zspan_last, zero_fn)
            plsc.subcore_barrier()

            def bbody(i, carry):
                b = s + NS * i
                boff = pl.multiple_of(b * EB, EB)
                pltpu.sync_copy(lst.at[pl.ds(boff, EB)], idxbuf)
                pltpu.sync_copy(onesbuf, dacc.at[idxbuf], add=True)
                return carry

            lax.fori_loop(0, nb, bbody, 0)
            plsc.subcore_barrier()

            def write_fn(r0, nr):
                pltpu.sync_copy(dacc.at[pl.ds(base + r0, nr)],
                                stage.at[pl.ds(0, nr)])
                pltpu.sync_copy(stage.at[pl.ds(0, nr)],
                                o.at[pl.ds(base + r0, nr)])

            _for_spans(spans_last, write_fn)
            plsc.subcore_barrier()


_deg_kernel = pl.kernel(
    _deg_body,
    out_type=(jax.ShapeDtypeStruct((DEG_N, CW), f32),) * 3,
    mesh=_sc_mesh(),
    scratch_types=[
        pltpu.VMEM((EB,), i32),
        pltpu.VMEM((EB, CW), f32),
        pltpu.VMEM((EB, CW), f32),
        pltpu.VMEM_SHARED((N + 8, CW), f32),
    ],
)


def _agg_body(table, ed2, z, out, pb0, pb1, r0b, r1b, acc,
              g0, g1, l0, l1):
    c = lax.axis_index("c")
    s = lax.axis_index("s")
    pbuf = (pb0, pb1)
    rows = (r0b, r1b)
    gsem = (g0, g1)
    lsem = (l0, l1)
    nb = jnp.where(s < 2, 79, 78)
    base = pl.multiple_of(s * 632, 8)

    def _rows_split(fn):
        @pl.when(s < 15)
        def _():
            fn(632)

        @pl.when(s == 15)
        def _():
            fn(520)

    for j in range(2):
        ch = 2 * c + j

        def zero_fn(nr):
            pltpu.sync_copy(z.at[pl.ds(0, nr)], acc.at[pl.ds(base, nr)])

        _rows_split(zero_fn)
        plsc.subcore_barrier()
        off = ch * N

        def _add_off(b):
            for v in range(8):
                pbuf[b][0, pl.ds(v * 16, 16)] = (
                    pbuf[b][0, pl.ds(v * 16, 16)] + off)

        pltpu.sync_copy(ed2.at[s], pb0)
        _add_off(0)
        pltpu.async_copy(table.at[pb0.at[0]], r0b, g0)
        pltpu.async_copy(ed2.at[s + NS], pb1, l1)

        def bbody(t, carry):
            for b in range(2):
                i = 2 * t + b

                @pl.when(i < nb)
                def _():
                    pltpu.make_async_copy(table.at[pbuf[b].at[0]], rows[b],
                                          gsem[b]).wait()

                    @pl.when(i + 1 < nb)
                    def _():
                        pltpu.make_async_copy(ed2.at[s], pbuf[1 - b],
                                              lsem[1 - b]).wait()
                        _add_off(1 - b)
                        pltpu.async_copy(table.at[pbuf[1 - b].at[0]],
                                         rows[1 - b], gsem[1 - b])

                    pltpu.sync_copy(rows[b], acc.at[pbuf[b].at[1]], add=True)

                    @pl.when(i + 2 < nb)
                    def _():
                        pltpu.async_copy(ed2.at[s + NS * (i + 2)],
                                         pbuf[b], lsem[b])
            return carry

        lax.fori_loop(0, 40, bbody, 0)
        plsc.subcore_barrier()
        obase = pl.multiple_of(ch * N, 8) + base

        def write_fn(nr):
            pltpu.sync_copy(acc.at[pl.ds(base, nr)],
                            out.at[pl.ds(obase, nr)])

        _rows_split(write_fn)
        plsc.subcore_barrier()


_agg_kernel = pl.kernel(
    _agg_body,
    out_type=jax.ShapeDtypeStruct((NCH * N, CW), f32),
    mesh=_sc_mesh(),
    scratch_types=[
        pltpu.VMEM((2, EB), i32),
        pltpu.VMEM((2, EB), i32),
        pltpu.VMEM((EB, CW), f32),
        pltpu.VMEM((EB, CW), f32),
        pltpu.VMEM_SHARED((N, CW), f32),
        pltpu.SemaphoreType.DMA,
        pltpu.SemaphoreType.DMA,
        pltpu.SemaphoreType.DMA,
        pltpu.SemaphoreType.DMA,
    ],
)


def _mm_body(a, w, b, o, *, nk):
    av = jnp.concatenate([a[k] for k in range(nk)], axis=1)
    acc = jnp.dot(av, w[...], preferred_element_type=f32)
    o[0] = acc + b[0]


def _mm(a3, w, b2):
    nk = a3.shape[0]
    return pl.pallas_call(
        functools.partial(_mm_body, nk=nk),
        grid=(N // BM, NCH),
        in_specs=[
            pl.BlockSpec((nk, BM, CW), lambda i, c: (0, i, 0)),
            pl.BlockSpec((nk * CW, CW), lambda i, c: (0, c)),
            pl.BlockSpec((1, 1, CW), lambda i, c: (c, 0, 0)),
        ],
        out_specs=pl.BlockSpec((1, BM, CW), lambda i, c: (c, i, 0)),
        out_shape=jax.ShapeDtypeStruct((NCH, N, CW), f32),
    )(a3, w, b2)


def _ew_op_body(hp, a1, d1, a3, d3, o):
    o[0] = jnp.maximum(hp[0] + a1[0] / jnp.maximum(d1[...], 1.0)
                       + a3[0] / jnp.maximum(d3[...], 1.0), 0.0)


def _ew_op(hp, a1, d1, a3, d3):
    return pl.pallas_call(
        _ew_op_body,
        grid=(N // BM, NCH),
        in_specs=[
            pl.BlockSpec((1, BM, CW), lambda i, c: (c, i, 0)),
            pl.BlockSpec((1, BM, CW), lambda i, c: (c, i, 0)),
            pl.BlockSpec((BM, CW), lambda i, c: (i, 0)),
            pl.BlockSpec((1, BM, CW), lambda i, c: (c, i, 0)),
            pl.BlockSpec((BM, CW), lambda i, c: (i, 0)),
        ],
        out_specs=pl.BlockSpec((1, BM, CW), lambda i, c: (c, i, 0)),
        out_shape=jax.ShapeDtypeStruct((NCH, N, CW), f32),
    )(hp, a1, d1, a3, d3)


def _ew_m_body(hp, a2, d2, o):
    o[0] = jnp.maximum(hp[0] + a2[0] / jnp.maximum(d2[...], 1.0), 0.0)


def _ew_m(hp, a2, d2):
    return pl.pallas_call(
        _ew_m_body,
        grid=(N // BM, NCH),
        in_specs=[
            pl.BlockSpec((1, BM, CW), lambda i, c: (c, i, 0)),
            pl.BlockSpec((1, BM, CW), lambda i, c: (c, i, 0)),
            pl.BlockSpec((BM, CW), lambda i, c: (i, 0)),
        ],
        out_specs=pl.BlockSpec((1, BM, CW), lambda i, c: (c, i, 0)),
        out_shape=jax.ShapeDtypeStruct((NCH, N, CW), f32),
    )(hp, a2, d2)


def _ew_sum_body(*args, nin):
    i = pl.program_id(1)
    if nin == 5:
        hp, a1, d1, a3, d3, osum = args
        h = jnp.maximum(hp[0] + a1[0] / jnp.maximum(d1[...], 1.0)
                        + a3[0] / jnp.maximum(d3[...], 1.0), 0.0)
    else:
        hp, a2, d2, osum = args
        h = jnp.maximum(hp[0] + a2[0] / jnp.maximum(d2[...], 1.0), 0.0)
    s2 = jnp.sum(h, axis=0, keepdims=True)

    @pl.when(i == 0)
    def _():
        osum[0] = s2

    @pl.when(i > 0)
    def _():
        osum[0] += s2


def _ew_sum(hp, aggs_and_degs):
    nin = 1 + len(aggs_and_degs)
    big = pl.BlockSpec((1, BM, CW), lambda c, i: (c, i, 0))
    deg = pl.BlockSpec((BM, CW), lambda c, i: (i, 0))
    specs = [big] + [big if k % 2 == 0 else deg
                     for k in range(len(aggs_and_degs))]
    return pl.pallas_call(
        functools.partial(_ew_sum_body, nin=nin),
        grid=(NCH, N // BM),
        in_specs=specs,
        out_specs=pl.BlockSpec((1, 1, CW), lambda c, i: (c, 0, 0)),
        out_shape=jax.ShapeDtypeStruct((NCH, 1, CW), f32),
    )(hp, *aggs_and_degs)


def _head_body(so, sm, w1, b1, w2, b2, o):
    acc = jnp.zeros((1, HID), f32)
    scale = 1.0 / N
    for k in range(NCH):
        acc += jnp.dot(so[pl.ds(k, 1), :] * scale, w1[pl.ds(k * CW, CW), :],
                       preferred_element_type=f32)
        acc += jnp.dot(sm[pl.ds(k, 1), :] * scale,
                       w1[pl.ds(HID + k * CW, CW), :],
                       preferred_element_type=f32)
    h = jnp.maximum(acc + b1[...], 0.0)
    o[...] = jnp.dot(h, w2[...], preferred_element_type=f32) + b2[...]


def _head(so, sm, w1, b1, w2p, b2p):
    return pl.pallas_call(
        _head_body,
        out_shape=jax.ShapeDtypeStruct((1, CW), f32),
    )(so, sm, w1, b1, w2p, b2p)


def kernel(X_op, X_m, E_seq, E_op2m, Wop0, bop0, Wm0, bm0, Wop1, bop1,
           Wm1, bm1, Wop2, bop2, Wm2, bm2, HW1, Hb1, HW2, Hb2):
    src_seq = E_seq[0]
    dst_seq = E_seq[1]
    src_op = E_op2m[0]
    dst_m = E_op2m[1]

    def _pair(a, b):
        return jnp.stack([a.reshape(NB_E, EB), b.reshape(NB_E, EB)], axis=1)

    ed_1 = _pair(src_seq, dst_seq)
    ed_2 = _pair(src_op, dst_m)
    ed_3 = _pair(dst_m, src_op)

    z128 = jnp.zeros((632, CW), f32)
    ones_h = jnp.ones((EB, CW), f32)
    zeros_h = jnp.zeros((EB, CW), f32)

    g1, g2, g3 = _deg_kernel(dst_seq, dst_m, src_op, ones_h, zeros_h)
    d1b = jnp.broadcast_to(g1[:N, :1], (N, CW))
    d2b = jnp.broadcast_to(g2[:N, :1], (N, CW))
    d3b = jnp.broadcast_to(g3[:N, :1], (N, CW))

    Hop = X_op.reshape(N, 2, CW).transpose(1, 0, 2)
    Hm = X_m.reshape(N, 2, CW).transpose(1, 0, 2)
    params = [(Wop0, bop0, Wm0, bm0), (Wop1, bop1, Wm1, bm1),
              (Wop2, bop2, Wm2, bm2)]

    for li, (Wo, bo, Wm_, bm_) in enumerate(params):
        Hp_op = _mm(Hop, Wo, bo.reshape(NCH, 1, CW))
        Hp_m = _mm(Hm, Wm_, bm_.reshape(NCH, 1, CW))
        tbl_op = Hp_op.reshape(NCH * N, CW)
        tbl_m = Hp_m.reshape(NCH * N, CW)
        a1 = _agg_kernel(tbl_op, ed_1, z128).reshape(NCH, N, CW)
        a2 = _agg_kernel(tbl_op, ed_2, z128).reshape(NCH, N, CW)
        a3 = _agg_kernel(tbl_m, ed_3, z128).reshape(NCH, N, CW)
        if li < 2:
            Hop = _ew_op(Hp_op, a1, d1b, a3, d3b)
            Hm = _ew_m(Hp_m, a2, d2b)
        else:
            s_op = _ew_sum(Hp_op, (a1, d1b, a3, d3b)).reshape(NCH, CW)
            s_m = _ew_sum(Hp_m, (a2, d2b)).reshape(NCH, CW)

    w2p = jnp.pad(HW2, ((0, 0), (0, CW - 1)))
    b2p = jnp.pad(Hb2, (0, CW - 1)).reshape(1, CW)
    y = _head(s_op, s_m, HW1, Hb1.reshape(1, HID), w2p, b2p)
    return y[0, :1]


# speedup vs baseline: 4.3474x; 1.0227x over previous
import functools

import jax
import jax.numpy as jnp
from jax import lax
from jax.experimental import pallas as pl
from jax.experimental.pallas import tpu as pltpu
from jax.experimental.pallas import tpu_sc as plsc

N = 10000
F_IN = 256
HID = 512
CW = 128
NCH = HID // CW
E = 160000
EB = 128
NB_E = E // EB
E_PAD = 163840
NBT = E_PAD // (16 * EB)
NS = 16
RPT = N // NS
DEG_N = 10240
BM = 1000

f32 = jnp.float32
i32 = jnp.int32


def _sc_mesh():
    return plsc.VectorSubcoreMesh(core_axis_name="c", subcore_axis_name="s",
                                  num_cores=2, num_subcores=NS)


def _deg_body(i1, i2, i3, ones_h, zeros_h, o1, o2, o3,
              ib0, ib1, onesbuf, stage, dacc, l0, l1):
    c = lax.axis_index("c")
    s = lax.axis_index("s")
    ibuf = (ib0, ib1)
    lsem = (l0, l1)
    pltpu.sync_copy(ones_h, onesbuf)
    nb = jnp.where(s < 2, 79, 78)
    base = pl.multiple_of(s * 632, 8)
    spans_main = ((0, 128), (128, 128), (256, 128), (384, 128), (512, 120))
    spans_last = ((0, 128), (128, 128), (256, 128), (384, 128), (512, 8))
    zspan_last = ((0, 128), (128, 128), (256, 128), (384, 128), (512, 16))

    def _for_spans(spl, fn):
        @pl.when(s < 15)
        def _():
            for r0, nr in spans_main:
                fn(r0, nr)

        @pl.when(s == 15)
        def _():
            for r0, nr in spl:
                fn(r0, nr)

    for lst, o, core in ((i1, o1, 0), (i2, o2, 1), (i3, o3, 0)):
        @pl.when(c == core)
        def _():
            pltpu.sync_copy(zeros_h, stage)

            def zero_fn(r0, nr):
                pltpu.sync_copy(stage.at[pl.ds(0, nr)],
                                acc_slice(r0, nr))

            def acc_slice(r0, nr):
                return dacc.at[pl.ds(base + r0, nr)]

            _for_spans(zspan_last, zero_fn)
            plsc.subcore_barrier()

            pltpu.async_copy(lst.at[pl.ds(pl.multiple_of(s * EB, EB), EB)],
                             ib0, l0)

            def bbody(t, carry):
                for b in range(2):
                    i = 2 * t + b

                    @pl.when(i < nb)
                    def _():
                        pltpu.make_async_copy(
                            lst.at[pl.ds(pl.multiple_of(s * EB, EB), EB)],
                            ibuf[b], lsem[b]).wait()

                        @pl.when(i + 1 < nb)
                        def _():
                            bo = pl.multiple_of((s + NS * (i + 1)) * EB, EB)
                            pltpu.async_copy(lst.at[pl.ds(bo, EB)],
                                             ibuf[1 - b], lsem[1 - b])

                        pltpu.sync_copy(onesbuf, dacc.at[ibuf[b]], add=True)
                return carry

            lax.fori_loop(0, 40, bbody, 0)
            plsc.subcore_barrier()

            def write_fn(r0, nr):
                pltpu.sync_copy(dacc.at[pl.ds(base + r0, nr)],
                                stage.at[pl.ds(0, nr)])
                pltpu.sync_copy(stage.at[pl.ds(0, nr)],
                                o.at[pl.ds(base + r0, nr)])

            _for_spans(spans_last, write_fn)
            plsc.subcore_barrier()


_deg_kernel = pl.kernel(
    _deg_body,
    out_type=(jax.ShapeDtypeStruct((DEG_N, CW), f32),) * 3,
    mesh=_sc_mesh(),
    scratch_types=[
        pltpu.VMEM((EB,), i32),
        pltpu.VMEM((EB,), i32),
        pltpu.VMEM((EB, CW), f32),
        pltpu.VMEM((EB, CW), f32),
        pltpu.VMEM_SHARED((N + 8, CW), f32),
        pltpu.SemaphoreType.DMA,
        pltpu.SemaphoreType.DMA,
    ],
)


def _agg_body(table, ed2, z, out, pb0, pb1, r0b, r1b, acc,
              g0, g1, l0, l1):
    c = lax.axis_index("c")
    s = lax.axis_index("s")
    pbuf = (pb0, pb1)
    rows = (r0b, r1b)
    gsem = (g0, g1)
    lsem = (l0, l1)
    nb = jnp.where(s < 2, 79, 78)
    base = pl.multiple_of(s * 632, 8)

    def _rows_split(fn):
        @pl.when(s < 15)
        def _():
            fn(632)

        @pl.when(s == 15)
        def _():
            fn(520)

    for j in range(2):
        ch = 2 * c + j

        def zero_fn(nr):
            pltpu.sync_copy(z.at[pl.ds(0, nr)], acc.at[pl.ds(base, nr)])

        _rows_split(zero_fn)
        plsc.subcore_barrier()
        off = ch * N

        def _add_off(b):
            for v in range(8):
                pbuf[b][0, pl.ds(v * 16, 16)] = (
                    pbuf[b][0, pl.ds(v * 16, 16)] + off)

        pltpu.sync_copy(ed2.at[s], pb0)
        _add_off(0)
        pltpu.async_copy(table.at[pb0.at[0]], r0b, g0)
        pltpu.async_copy(ed2.at[s + NS], pb1, l1)

        def bbody(t, carry):
            for b in range(2):
                i = 2 * t + b

                @pl.when(i < nb)
                def _():
                    pltpu.make_async_copy(table.at[pbuf[b].at[0]], rows[b],
                                          gsem[b]).wait()

                    @pl.when(i + 1 < nb)
                    def _():
                        pltpu.make_async_copy(ed2.at[s], pbuf[1 - b],
                                              lsem[1 - b]).wait()
                        _add_off(1 - b)
                        pltpu.async_copy(table.at[pbuf[1 - b].at[0]],
                                         rows[1 - b], gsem[1 - b])

                    pltpu.sync_copy(rows[b], acc.at[pbuf[b].at[1]], add=True)

                    @pl.when(i + 2 < nb)
                    def _():
                        pltpu.async_copy(ed2.at[s + NS * (i + 2)],
                                         pbuf[b], lsem[b])
            return carry

        lax.fori_loop(0, 40, bbody, 0)
        plsc.subcore_barrier()
        obase = pl.multiple_of(ch * N, 8) + base

        def write_fn(nr):
            pltpu.sync_copy(acc.at[pl.ds(base, nr)],
                            out.at[pl.ds(obase, nr)])

        _rows_split(write_fn)
        plsc.subcore_barrier()


_agg_kernel = pl.kernel(
    _agg_body,
    out_type=jax.ShapeDtypeStruct((NCH * N, CW), f32),
    mesh=_sc_mesh(),
    scratch_types=[
        pltpu.VMEM((2, EB), i32),
        pltpu.VMEM((2, EB), i32),
        pltpu.VMEM((EB, CW), f32),
        pltpu.VMEM((EB, CW), f32),
        pltpu.VMEM_SHARED((N, CW), f32),
        pltpu.SemaphoreType.DMA,
        pltpu.SemaphoreType.DMA,
        pltpu.SemaphoreType.DMA,
        pltpu.SemaphoreType.DMA,
    ],
)


def _mm_body(a, w, b, o, *, nk):
    av = jnp.concatenate([a[k] for k in range(nk)], axis=1)
    acc = jnp.dot(av, w[...], preferred_element_type=f32)
    o[0] = acc + b[0]


def _mm(a3, w, b2):
    nk = a3.shape[0]
    return pl.pallas_call(
        functools.partial(_mm_body, nk=nk),
        grid=(N // BM, NCH),
        in_specs=[
            pl.BlockSpec((nk, BM, CW), lambda i, c: (0, i, 0)),
            pl.BlockSpec((nk * CW, CW), lambda i, c: (0, c)),
            pl.BlockSpec((1, 1, CW), lambda i, c: (c, 0, 0)),
        ],
        out_specs=pl.BlockSpec((1, BM, CW), lambda i, c: (c, i, 0)),
        out_shape=jax.ShapeDtypeStruct((NCH, N, CW), f32),
    )(a3, w, b2)


def _ew_op_body(hp, a1, d1, a3, d3, o):
    o[0] = jnp.maximum(hp[0] + a1[0] / jnp.maximum(d1[...], 1.0)
                       + a3[0] / jnp.maximum(d3[...], 1.0), 0.0)


def _ew_op(hp, a1, d1, a3, d3):
    return pl.pallas_call(
        _ew_op_body,
        grid=(N // BM, NCH),
        in_specs=[
            pl.BlockSpec((1, BM, CW), lambda i, c: (c, i, 0)),
            pl.BlockSpec((1, BM, CW), lambda i, c: (c, i, 0)),
            pl.BlockSpec((BM, CW), lambda i, c: (i, 0)),
            pl.BlockSpec((1, BM, CW), lambda i, c: (c, i, 0)),
            pl.BlockSpec((BM, CW), lambda i, c: (i, 0)),
        ],
        out_specs=pl.BlockSpec((1, BM, CW), lambda i, c: (c, i, 0)),
        out_shape=jax.ShapeDtypeStruct((NCH, N, CW), f32),
    )(hp, a1, d1, a3, d3)


def _ew_m_body(hp, a2, d2, o):
    o[0] = jnp.maximum(hp[0] + a2[0] / jnp.maximum(d2[...], 1.0), 0.0)


def _ew_m(hp, a2, d2):
    return pl.pallas_call(
        _ew_m_body,
        grid=(N // BM, NCH),
        in_specs=[
            pl.BlockSpec((1, BM, CW), lambda i, c: (c, i, 0)),
            pl.BlockSpec((1, BM, CW), lambda i, c: (c, i, 0)),
            pl.BlockSpec((BM, CW), lambda i, c: (i, 0)),
        ],
        out_specs=pl.BlockSpec((1, BM, CW), lambda i, c: (c, i, 0)),
        out_shape=jax.ShapeDtypeStruct((NCH, N, CW), f32),
    )(hp, a2, d2)


def _ew_sum_body(*args, nin):
    i = pl.program_id(1)
    if nin == 5:
        hp, a1, d1, a3, d3, osum = args
        h = jnp.maximum(hp[0] + a1[0] / jnp.maximum(d1[...], 1.0)
                        + a3[0] / jnp.maximum(d3[...], 1.0), 0.0)
    else:
        hp, a2, d2, osum = args
        h = jnp.maximum(hp[0] + a2[0] / jnp.maximum(d2[...], 1.0), 0.0)
    s2 = jnp.sum(h, axis=0, keepdims=True)

    @pl.when(i == 0)
    def _():
        osum[0] = s2

    @pl.when(i > 0)
    def _():
        osum[0] += s2


def _ew_sum(hp, aggs_and_degs):
    nin = 1 + len(aggs_and_degs)
    big = pl.BlockSpec((1, BM, CW), lambda c, i: (c, i, 0))
    deg = pl.BlockSpec((BM, CW), lambda c, i: (i, 0))
    specs = [big] + [big if k % 2 == 0 else deg
                     for k in range(len(aggs_and_degs))]
    return pl.pallas_call(
        functools.partial(_ew_sum_body, nin=nin),
        grid=(NCH, N // BM),
        in_specs=specs,
        out_specs=pl.BlockSpec((1, 1, CW), lambda c, i: (c, 0, 0)),
        out_shape=jax.ShapeDtypeStruct((NCH, 1, CW), f32),
    )(hp, *aggs_and_degs)


def _head_body(so, sm, w1, b1, w2, b2, o):
    acc = jnp.zeros((1, HID), f32)
    scale = 1.0 / N
    for k in range(NCH):
        acc += jnp.dot(so[pl.ds(k, 1), :] * scale, w1[pl.ds(k * CW, CW), :],
                       preferred_element_type=f32)
        acc += jnp.dot(sm[pl.ds(k, 1), :] * scale,
                       w1[pl.ds(HID + k * CW, CW), :],
                       preferred_element_type=f32)
    h = jnp.maximum(acc + b1[...], 0.0)
    o[...] = jnp.dot(h, w2[...], preferred_element_type=f32) + b2[...]


def _head(so, sm, w1, b1, w2p, b2p):
    return pl.pallas_call(
        _head_body,
        out_shape=jax.ShapeDtypeStruct((1, CW), f32),
    )(so, sm, w1, b1, w2p, b2p)


def kernel(X_op, X_m, E_seq, E_op2m, Wop0, bop0, Wm0, bm0, Wop1, bop1,
           Wm1, bm1, Wop2, bop2, Wm2, bm2, HW1, Hb1, HW2, Hb2):
    src_seq = E_seq[0]
    dst_seq = E_seq[1]
    src_op = E_op2m[0]
    dst_m = E_op2m[1]

    def _pair(a, b):
        return jnp.stack([a.reshape(NB_E, EB), b.reshape(NB_E, EB)], axis=1)

    ed_1 = _pair(src_seq, dst_seq)
    ed_2 = _pair(src_op, dst_m)
    ed_3 = _pair(dst_m, src_op)

    z128 = jnp.zeros((632, CW), f32)
    ones_h = jnp.ones((EB, CW), f32)
    zeros_h = jnp.zeros((EB, CW), f32)

    g1, g2, g3 = _deg_kernel(dst_seq, dst_m, src_op, ones_h, zeros_h)
    d1b = jnp.broadcast_to(g1[:N, :1], (N, CW))
    d2b = jnp.broadcast_to(g2[:N, :1], (N, CW))
    d3b = jnp.broadcast_to(g3[:N, :1], (N, CW))

    Hop = X_op.reshape(N, 2, CW).transpose(1, 0, 2)
    Hm = X_m.reshape(N, 2, CW).transpose(1, 0, 2)
    params = [(Wop0, bop0, Wm0, bm0), (Wop1, bop1, Wm1, bm1),
              (Wop2, bop2, Wm2, bm2)]

    for li, (Wo, bo, Wm_, bm_) in enumerate(params):
        Hp_op = _mm(Hop, Wo, bo.reshape(NCH, 1, CW))
        Hp_m = _mm(Hm, Wm_, bm_.reshape(NCH, 1, CW))
        tbl_op = Hp_op.reshape(NCH * N, CW)
        tbl_m = Hp_m.reshape(NCH * N, CW)
        a1 = _agg_kernel(tbl_op, ed_1, z128).reshape(NCH, N, CW)
        a2 = _agg_kernel(tbl_op, ed_2, z128).reshape(NCH, N, CW)
        a3 = _agg_kernel(tbl_m, ed_3, z128).reshape(NCH, N, CW)
        if li < 2:
            Hop = _ew_op(Hp_op, a1, d1b, a3, d3b)
            Hm = _ew_m(Hp_m, a2, d2b)
        else:
            s_op = _ew_sum(Hp_op, (a1, d1b, a3, d3b)).reshape(NCH, CW)
            s_m = _ew_sum(Hp_m, (a2, d2b)).reshape(NCH, CW)

    w2p = jnp.pad(HW2, ((0, 0), (0, CW - 1)))
    b2p = jnp.pad(Hb2, (0, CW - 1)).reshape(1, CW)
    y = _head(s_op, s_m, HW1, Hb1.reshape(1, HID), w2p, b2p)
    return y[0, :1]
